# baseline (device time: 276393 ns/iter reference)
import jax
import jax.numpy as jnp
from jax import lax
from jax.experimental import pallas as pl
from jax.experimental.pallas import tpu as pltpu

N_DEV = 16
COMM_DTYPE = jnp.bfloat16
N_SUB = 1


def kernel(x, w_mat, scale_x, scale_w):
    m, k_shard = x.shape
    _, n_out = w_mat.shape
    chunk = m // N_DEV
    n_lanes = 2 * N_SUB
    sub = n_out // n_lanes
    n_hops = 2 * (N_DEV - 1)

    def body(x_ref, w_ref, sx_ref, sw_ref, out_ref,
             comm_ref, stage_ref, send_sems, recv_sems, wb_ref, credits):
        my = lax.axis_index("i")
        left = (my - 1) % N_DEV
        right = (my + 1) % N_DEV

        lanes = []
        for li in range(n_lanes):
            d = 1 if li < N_SUB else -1
            to_peer = right if d == 1 else left
            up_peer = left if d == 1 else right
            lanes.append((li * sub, to_peer, up_peer, d))

        barrier = pltpu.get_barrier_semaphore()
        for nbr in (left, right):
            pl.semaphore_signal(barrier, inc=1, device_id=(nbr,),
                                device_id_type=pl.DeviceIdType.MESH)
        pl.semaphore_wait(barrier, 2)

        wb_ref[...] = w_ref[...].astype(jnp.bfloat16)
        s = sx_ref[0] * sw_ref[0]

        def partial(c, lo):
            xa = x_ref[pl.ds(c * chunk, chunk), :].astype(jnp.bfloat16)
            return lax.dot_general(
                xa, wb_ref[:, lo:lo + sub],
                dimension_numbers=(((1,), (0,)), ((), ())),
                preferred_element_type=jnp.float32)

        def silu(v):
            y = v * s
            return y * (1.0 / (1.0 + jnp.exp(-y)))

        def desc(li, slot, peer):
            return pltpu.make_async_remote_copy(
                src_ref=stage_ref.at[li, slot],
                dst_ref=comm_ref.at[li, slot],
                send_sem=send_sems.at[li, slot],
                recv_sem=recv_sems.at[li, slot],
                device_id=(peer,),
                device_id_type=pl.DeviceIdType.MESH,
            )

        for li, (lo, to_peer, _up, _d) in enumerate(lanes):
            stage_ref[li, 0] = partial(my, lo).astype(COMM_DTYPE)
            desc(li, 0, to_peer).start()

        for h in range(n_hops):
            slot = h % 2
            nslot = (h + 1) % 2
            rs = h < N_DEV - 1
            g = h - (N_DEV - 1)
            for li, (lo, to_peer, up_peer, d) in enumerate(lanes):
                c_recv = (my - d * (1 + h if rs else g)) % N_DEV

                if h >= 1:
                    desc(li, nslot, to_peer).wait_send()
                desc(li, slot, to_peer).wait_recv()

                if rs:
                    val = (comm_ref[li, slot].astype(jnp.float32)
                           + partial(c_recv, lo))
                    if h == N_DEV - 2:
                        val = silu(val)
                        out_ref[pl.ds(c_recv * chunk, chunk),
                                lo:lo + sub] = val
                    stage_ref[li, nslot] = val.astype(COMM_DTYPE)
                else:
                    out_ref[pl.ds(c_recv * chunk, chunk), lo:lo + sub] = (
                        comm_ref[li, slot].astype(jnp.float32))
                    if h < n_hops - 1:
                        stage_ref[li, nslot] = comm_ref[li, slot]

                if h < n_hops - 1:
                    if h + 1 >= 2:
                        pl.semaphore_wait(credits.at[li], 1)
                    desc(li, nslot, to_peer).start()
                pl.semaphore_signal(credits.at[li], inc=1,
                                    device_id=(up_peer,),
                                    device_id_type=pl.DeviceIdType.MESH)

        for li, (_lo, to_peer, _up, _d) in enumerate(lanes):
            desc(li, (n_hops - 1) % 2, to_peer).wait_send()
            pl.semaphore_wait(credits.at[li], 2)

    return pl.pallas_call(
        body,
        out_shape=jax.ShapeDtypeStruct((m, n_out), jnp.float32),
        in_specs=[
            pl.BlockSpec(memory_space=pltpu.VMEM),
            pl.BlockSpec(memory_space=pltpu.VMEM),
            pl.BlockSpec(memory_space=pltpu.SMEM),
            pl.BlockSpec(memory_space=pltpu.SMEM),
        ],
        out_specs=pl.BlockSpec(memory_space=pltpu.VMEM),
        scratch_shapes=[
            pltpu.VMEM((n_lanes, 2, chunk, sub), COMM_DTYPE),
            pltpu.VMEM((n_lanes, 2, chunk, sub), COMM_DTYPE),
            pltpu.SemaphoreType.DMA((n_lanes, 2)),
            pltpu.SemaphoreType.DMA((n_lanes, 2)),
            pltpu.VMEM((k_shard, n_out), jnp.bfloat16),
            pltpu.SemaphoreType.REGULAR((n_lanes,)),
        ],
        compiler_params=pltpu.CompilerParams(
            collective_id=0,
            vmem_limit_bytes=100 * 1024 * 1024,
        ),
    )(x, w_mat, scale_x, scale_w)


# device time: 275995 ns/iter; 1.0014x vs baseline; 1.0014x over previous
import jax
import jax.numpy as jnp
from jax import lax
from jax.experimental import pallas as pl
from jax.experimental.pallas import tpu as pltpu

N_DEV = 16
N_HOPS = N_DEV - 1
DOT_MODE = "post"
N_GROUPS = 4
N_PANELS = 16


def kernel(x, w_mat, scale_x, scale_w):
    m, k_shard = x.shape
    _, n_out = w_mat.shape
    kh = k_shard // 2
    kg = kh * N_DEV // N_GROUPS
    rows = m // N_PANELS
    panels_per_hop = N_PANELS // 4

    def body(x_ref, w_ref, sx_ref, sw_ref, out_ref,
             xg_r, wg_r, xg_l, wg_l, send_sems, recv_sems):
        my = lax.axis_index("i")
        left = (my - 1) % N_DEV
        right = (my + 1) % N_DEV

        lanes = [(True, xg_r, right), (False, wg_r, right),
                 (True, xg_l, left), (False, wg_l, left)]

        def xslice(buf, s):
            return buf.at[:, s * kh:(s + 1) * kh]

        def wslice(buf, s):
            return buf.at[s * kh:(s + 1) * kh, :]

        def desc(li, h):
            is_x, buf, peer = lanes[li]
            sl = xslice if is_x else wslice
            return pltpu.make_async_remote_copy(
                src_ref=sl(buf, h),
                dst_ref=sl(buf, h + 1),
                send_sem=send_sems.at[li],
                recv_sem=recv_sems.at[li, h],
                device_id=(peer,),
                device_id_type=pl.DeviceIdType.MESH,
            )

        xg_r[:, 0:kh] = x_ref[:, 0:kh]
        wg_r[0:kh, :] = w_ref[0:kh, :]
        xg_l[:, 0:kh] = x_ref[:, kh:2 * kh]
        wg_l[0:kh, :] = w_ref[kh:2 * kh, :]

        s = sx_ref[0] * sw_ref[0]

        def panel_dots(g, p0):
            k0 = g * kg

            def one_panel(p, carry):
                r0 = p * rows
                a = lax.dot_general(
                    xg_r[pl.ds(r0, rows), k0:k0 + kg], wg_r[k0:k0 + kg, :],
                    dimension_numbers=(((1,), (0,)), ((), ())),
                    preferred_element_type=jnp.float32)
                b = lax.dot_general(
                    xg_l[pl.ds(r0, rows), k0:k0 + kg], wg_l[k0:k0 + kg, :],
                    dimension_numbers=(((1,), (0,)), ((), ())),
                    preferred_element_type=jnp.float32)
                if g == 0:
                    out_ref[pl.ds(r0, rows), :] = a + b
                elif g == N_GROUPS - 1:
                    y = (out_ref[pl.ds(r0, rows), :] + a + b) * s
                    out_ref[pl.ds(r0, rows), :] = (
                        y * (1.0 / (1.0 + jnp.exp(-y))))
                else:
                    out_ref[pl.ds(r0, rows), :] = (
                        out_ref[pl.ds(r0, rows), :] + a + b)
                return carry

            lax.fori_loop(p0, p0 + panels_per_hop, one_panel, 0)

        due = {}
        for g in range(N_GROUPS):
            for w in range(N_PANELS // panels_per_hop):
                due[4 * g + 2 + w] = (g, w * panels_per_hop)

        barrier = pltpu.get_barrier_semaphore()
        for nbr in (left, right):
            pl.semaphore_signal(barrier, inc=1, device_id=(nbr,),
                                device_id_type=pl.DeviceIdType.MESH)
        pl.semaphore_wait(barrier, 2)

        for h in range(N_HOPS):
            for li in range(4):
                if h >= 1:
                    desc(li, h - 1).wait_send()
                desc(li, h).start()
            for li in range(4):
                desc(li, h).wait_recv()
            if h in due and DOT_MODE == "inline":
                panel_dots(*due[h])

        for li in range(4):
            desc(li, N_HOPS - 1).wait_send()
        if DOT_MODE == "inline":
            for h in range(N_HOPS, max(due) + 1):
                panel_dots(*due[h])
        elif DOT_MODE == "post":
            for h in sorted(due):
                panel_dots(*due[h])
        else:
            panel_dots(0, 0)

    x8 = x.astype(jnp.float8_e4m3fn)
    w8 = w_mat.astype(jnp.float8_e5m2)

    return pl.pallas_call(
        body,
        out_shape=jax.ShapeDtypeStruct((m, n_out), jnp.float32),
        in_specs=[
            pl.BlockSpec(memory_space=pltpu.VMEM),
            pl.BlockSpec(memory_space=pltpu.VMEM),
            pl.BlockSpec(memory_space=pltpu.SMEM),
            pl.BlockSpec(memory_space=pltpu.SMEM),
        ],
        out_specs=pl.BlockSpec(memory_space=pltpu.VMEM),
        scratch_shapes=[
            pltpu.VMEM((m, kh * N_DEV), jnp.float8_e4m3fn),
            pltpu.VMEM((kh * N_DEV, n_out), jnp.float8_e5m2),
            pltpu.VMEM((m, kh * N_DEV), jnp.float8_e4m3fn),
            pltpu.VMEM((kh * N_DEV, n_out), jnp.float8_e5m2),
            pltpu.SemaphoreType.DMA((4,)),
            pltpu.SemaphoreType.DMA((4, N_HOPS)),
        ],
        compiler_params=pltpu.CompilerParams(
            collective_id=0,
            vmem_limit_bytes=100 * 1024 * 1024,
        ),
    )(x8, w8, scale_x, scale_w)


# device time: 230062 ns/iter; 1.2014x vs baseline; 1.1997x over previous
import jax
import jax.numpy as jnp
from jax import lax
from jax.experimental import pallas as pl
from jax.experimental.pallas import tpu as pltpu

N_DEV = 16
COMM_DTYPE = jnp.bfloat16
N_SUB = 4


def kernel(x, w_mat, scale_x, scale_w):
    m, k_shard = x.shape
    _, n_out = w_mat.shape
    chunk = m // N_DEV
    n_lanes = 2 * N_SUB
    sub = n_out // n_lanes
    n_hops = 2 * (N_DEV - 1)

    def body(x_ref, w_ref, sx_ref, sw_ref, out_ref,
             comm_ref, stage_ref, send_sems, recv_sems, wb_ref, credits):
        my = lax.axis_index("i")
        left = (my - 1) % N_DEV
        right = (my + 1) % N_DEV

        lanes = []
        for li in range(n_lanes):
            d = 1 if li < N_SUB else -1
            to_peer = right if d == 1 else left
            up_peer = left if d == 1 else right
            lanes.append((li * sub, to_peer, up_peer, d))

        barrier = pltpu.get_barrier_semaphore()
        for nbr in (left, right):
            pl.semaphore_signal(barrier, inc=1, device_id=(nbr,),
                                device_id_type=pl.DeviceIdType.MESH)
        pl.semaphore_wait(barrier, 2)

        wb_ref[...] = w_ref[...].astype(jnp.bfloat16)
        s = sx_ref[0] * sw_ref[0]

        def partial(c, lo):
            xa = x_ref[pl.ds(c * chunk, chunk), :].astype(jnp.bfloat16)
            return lax.dot_general(
                xa, wb_ref[:, lo:lo + sub],
                dimension_numbers=(((1,), (0,)), ((), ())),
                preferred_element_type=jnp.float32)

        def silu(v):
            y = v * s
            return y * (1.0 / (1.0 + jnp.exp(-y)))

        def desc(li, slot, peer):
            return pltpu.make_async_remote_copy(
                src_ref=stage_ref.at[li, slot],
                dst_ref=comm_ref.at[li, slot],
                send_sem=send_sems.at[li, slot],
                recv_sem=recv_sems.at[li, slot],
                device_id=(peer,),
                device_id_type=pl.DeviceIdType.MESH,
            )

        for li, (lo, to_peer, _up, _d) in enumerate(lanes):
            stage_ref[li, 0] = partial(my, lo).astype(COMM_DTYPE)
            desc(li, 0, to_peer).start()

        for h in range(n_hops):
            slot = h % 2
            nslot = (h + 1) % 2
            rs = h < N_DEV - 1
            g = h - (N_DEV - 1)
            for li, (lo, to_peer, up_peer, d) in enumerate(lanes):
                c_recv = (my - d * (1 + h if rs else g)) % N_DEV

                if h >= 1:
                    desc(li, nslot, to_peer).wait_send()
                desc(li, slot, to_peer).wait_recv()

                if rs:
                    val = (comm_ref[li, slot].astype(jnp.float32)
                           + partial(c_recv, lo))
                    if h == N_DEV - 2:
                        val = silu(val)
                        out_ref[pl.ds(c_recv * chunk, chunk),
                                lo:lo + sub] = val
                    stage_ref[li, nslot] = val.astype(COMM_DTYPE)
                else:
                    out_ref[pl.ds(c_recv * chunk, chunk), lo:lo + sub] = (
                        comm_ref[li, slot].astype(jnp.float32))
                    if h < n_hops - 1:
                        stage_ref[li, nslot] = comm_ref[li, slot]

                if h < n_hops - 1:
                    if h + 1 >= 2:
                        pl.semaphore_wait(credits.at[li], 1)
                    desc(li, nslot, to_peer).start()
                pl.semaphore_signal(credits.at[li], inc=1,
                                    device_id=(up_peer,),
                                    device_id_type=pl.DeviceIdType.MESH)

        for li, (_lo, to_peer, _up, _d) in enumerate(lanes):
            desc(li, (n_hops - 1) % 2, to_peer).wait_send()
            pl.semaphore_wait(credits.at[li], 2)

    return pl.pallas_call(
        body,
        out_shape=jax.ShapeDtypeStruct((m, n_out), jnp.float32),
        in_specs=[
            pl.BlockSpec(memory_space=pltpu.VMEM),
            pl.BlockSpec(memory_space=pltpu.VMEM),
            pl.BlockSpec(memory_space=pltpu.SMEM),
            pl.BlockSpec(memory_space=pltpu.SMEM),
        ],
        out_specs=pl.BlockSpec(memory_space=pltpu.VMEM),
        scratch_shapes=[
            pltpu.VMEM((n_lanes, 2, chunk, sub), COMM_DTYPE),
            pltpu.VMEM((n_lanes, 2, chunk, sub), COMM_DTYPE),
            pltpu.SemaphoreType.DMA((n_lanes, 2)),
            pltpu.SemaphoreType.DMA((n_lanes, 2)),
            pltpu.VMEM((k_shard, n_out), jnp.bfloat16),
            pltpu.SemaphoreType.REGULAR((n_lanes,)),
        ],
        compiler_params=pltpu.CompilerParams(
            collective_id=0,
            vmem_limit_bytes=100 * 1024 * 1024,
        ),
    )(x, w_mat, scale_x, scale_w)
